# baseline (speedup 1.0000x reference)
import numpy as np
import jax
import jax.numpy as jnp
from jax.experimental import pallas as pl

_B = 128
_V = 100000
_RB = 8
_NB = _B // _RB

_KS0 = np.uint32(0)
_KS1 = np.uint32(42)
_KS2 = np.uint32(np.uint32(0x1BD11BDA) ^ np.uint32(42))
_TINY = np.float32(np.finfo(np.float32).tiny)
_SCALE = np.float32(1.0 - np.finfo(np.float32).tiny)
_INT_MAX = np.int32(2**31 - 1)


def _rotl(x, r):
    return jax.lax.shift_left(x, np.uint32(r)) | jax.lax.shift_right_logical(
        x, np.uint32(32 - r)
    )


def _threefry_xor_bits(lin):
    x0 = jnp.zeros_like(lin)
    x1 = lin + _KS1
    ks = (_KS0, _KS1, _KS2)
    rots = ((13, 15, 26, 6), (17, 29, 16, 24))
    for i in range(5):
        for r in rots[i % 2]:
            x0 = x0 + x1
            x1 = _rotl(x1, r) ^ x0
        x0 = x0 + ks[(i + 1) % 3]
        x1 = x1 + ks[(i + 2) % 3] + np.uint32(i + 1)
    return x0 ^ x1


def _main_kernel(p_ref, out_ref, res_ref):
    b = pl.program_id(0)
    p = p_ref[...]

    r_i = jax.lax.broadcasted_iota(jnp.uint32, (_RB, _V), 0)
    c_i = jax.lax.broadcasted_iota(jnp.uint32, (_RB, _V), 1)
    lin = (b.astype(jnp.uint32) * np.uint32(_RB) + r_i) * np.uint32(_V) + c_i
    bits = _threefry_xor_bits(lin)

    fb = jax.lax.bitcast_convert_type(
        (bits >> np.uint32(9)) | np.uint32(0x3F800000), jnp.float32
    ) - np.float32(1.0)
    u = jnp.maximum(_TINY, fb * _SCALE + _TINY)
    val = -jnp.log(-jnp.log(u)) + jnp.log(jnp.maximum(p, np.float32(1e-30)))

    bm = jnp.max(val, axis=1, keepdims=True)
    col = jax.lax.broadcasted_iota(jnp.int32, (_RB, _V), 1)
    bidx = jnp.min(jnp.where(val == bm, col, _INT_MAX), axis=1, keepdims=True)

    res_ref[...] = bidx
    out_ref[...] = jnp.where(
        col == bidx,
        (np.float32(1.0) - p) / jnp.maximum(p, np.float32(1e-10)),
        p / jnp.maximum(np.float32(1.0) - p, np.float32(1e-10)),
    )


def kernel(p, alpha, beta):
    del alpha, beta
    unc, res = pl.pallas_call(
        _main_kernel,
        grid=(_NB,),
        in_specs=[pl.BlockSpec((_RB, _V), lambda b: (b, 0))],
        out_specs=[
            pl.BlockSpec((_RB, _V), lambda b: (b, 0)),
            pl.BlockSpec((_RB, 1), lambda b: (b, 0)),
        ],
        out_shape=[
            jax.ShapeDtypeStruct((_B, _V), jnp.float32),
            jax.ShapeDtypeStruct((_B, 1), jnp.int32),
        ],
    )(p)
    return (res, unc)


# speedup vs baseline: 1.5502x; 1.5502x over previous
import numpy as np
import jax
import jax.numpy as jnp
from jax.experimental import pallas as pl

_B = 128
_V = 100000
_RB = 8
_NB = _B // _RB
_CW = 1024
_NFULL = _V // _CW
_REM = _V - _NFULL * _CW

_KS0 = np.uint32(0)
_KS1 = np.uint32(42)
_KS2 = np.uint32(np.uint32(0x1BD11BDA) ^ np.uint32(42))
_TINY = np.float32(np.finfo(np.float32).tiny)
_SCALE = np.float32(1.0 - np.finfo(np.float32).tiny)
_INT_MAX = np.int32(2**31 - 1)


def _rotl(x, r):
    return jax.lax.shift_left(x, np.uint32(r)) | jax.lax.shift_right_logical(
        x, np.uint32(32 - r)
    )


def _threefry_xor_bits(lin):
    x0 = jnp.zeros_like(lin)
    x1 = lin + _KS1
    ks = (_KS0, _KS1, _KS2)
    rots = ((13, 15, 26, 6), (17, 29, 16, 24))
    for i in range(5):
        for r in rots[i % 2]:
            x0 = x0 + x1
            x1 = _rotl(x1, r) ^ x0
        x0 = x0 + ks[(i + 1) % 3]
        x1 = x1 + ks[(i + 2) % 3] + np.uint32(i + 1)
    return x0 ^ x1


def _perturbed(p, lin):
    bits = _threefry_xor_bits(lin)
    fb = jax.lax.bitcast_convert_type(
        (bits >> np.uint32(9)) | np.uint32(0x3F800000), jnp.float32
    ) - np.float32(1.0)
    u = jnp.maximum(_TINY, fb * _SCALE + _TINY)
    return -jnp.log(-jnp.log(u)) + jnp.log(jnp.maximum(p, np.float32(1e-30)))


def _main_kernel(p_ref, out_ref, res_ref):
    b = pl.program_id(0)

    sub = jax.lax.broadcasted_iota(jnp.uint32, (_RB, _CW), 0)
    rowbase = (b.astype(jnp.uint32) * np.uint32(_RB) + sub) * np.uint32(_V)
    lane = jax.lax.broadcasted_iota(jnp.uint32, (_RB, _CW), 1)
    lane_i = jax.lax.broadcasted_iota(jnp.int32, (_RB, _CW), 1)

    m_lane = None
    for k in range(_NFULL):
        off = k * _CW
        p = p_ref[:, pl.ds(off, _CW)]
        val = _perturbed(p, rowbase + lane + np.uint32(off))
        col = lane_i + np.int32(off)
        if m_lane is None:
            m_lane, col_lane, p_lane = val, col, p
        else:
            upd = val > m_lane
            m_lane = jnp.where(upd, val, m_lane)
            col_lane = jnp.where(upd, col, col_lane)
            p_lane = jnp.where(upd, p, p_lane)

    off = _NFULL * _CW
    p_r = p_ref[:, pl.ds(off, _REM)]
    val_r = _perturbed(
        p_r,
        (b.astype(jnp.uint32) * np.uint32(_RB)
         + jax.lax.broadcasted_iota(jnp.uint32, (_RB, _REM), 0))
        * np.uint32(_V)
        + jax.lax.broadcasted_iota(jnp.uint32, (_RB, _REM), 1)
        + np.uint32(off),
    )
    col_r = jax.lax.broadcasted_iota(jnp.int32, (_RB, _REM), 1) + np.int32(off)

    bm = jnp.maximum(
        jnp.max(m_lane, axis=1, keepdims=True),
        jnp.max(val_r, axis=1, keepdims=True),
    )
    idx = jnp.minimum(
        jnp.min(jnp.where(m_lane == bm, col_lane, _INT_MAX), axis=1, keepdims=True),
        jnp.min(jnp.where(val_r == bm, col_r, _INT_MAX), axis=1, keepdims=True),
    )
    p_at = jnp.maximum(
        jnp.max(
            jnp.where((m_lane == bm) & (col_lane == idx), p_lane, np.float32(-1.0)),
            axis=1, keepdims=True,
        ),
        jnp.max(
            jnp.where((val_r == bm) & (col_r == idx), p_r, np.float32(-1.0)),
            axis=1, keepdims=True,
        ),
    )
    vc = (np.float32(1.0) - p_at) / jnp.maximum(p_at, np.float32(1e-10))

    res_ref[...] = idx

    for k in range(_NFULL + 1):
        off = k * _CW
        w = _CW if k < _NFULL else _REM
        p = p_ref[:, pl.ds(off, w)]
        col = (jax.lax.broadcasted_iota(jnp.int32, (_RB, w), 1) + np.int32(off)
               if w != _CW else lane_i + np.int32(off))
        vnc = p / jnp.maximum(np.float32(1.0) - p, np.float32(1e-10))
        out_ref[:, pl.ds(off, w)] = jnp.where(col == idx, vc, vnc)


def kernel(p, alpha, beta):
    del alpha, beta
    unc, res = pl.pallas_call(
        _main_kernel,
        grid=(_NB,),
        in_specs=[pl.BlockSpec((_RB, _V), lambda b: (b, 0))],
        out_specs=[
            pl.BlockSpec((_RB, _V), lambda b: (b, 0)),
            pl.BlockSpec((_RB, 1), lambda b: (b, 0)),
        ],
        out_shape=[
            jax.ShapeDtypeStruct((_B, _V), jnp.float32),
            jax.ShapeDtypeStruct((_B, 1), jnp.int32),
        ],
    )(p)
    return (res, unc)


# speedup vs baseline: 1.8010x; 1.1617x over previous
import numpy as np
import jax
import jax.numpy as jnp
from jax.experimental import pallas as pl
from jax.experimental.pallas import tpu as pltpu

_B = 128
_V = 100000
_BV = 4000
_NS = _V // _BV
_CH = 160
_NCH = _BV // _CH

_KS0 = np.uint32(0)
_KS1 = np.uint32(42)
_KS2 = np.uint32(np.uint32(0x1BD11BDA) ^ np.uint32(42))
_TINY = np.float32(np.finfo(np.float32).tiny)
_SCALE = np.float32(1.0 - np.finfo(np.float32).tiny)
_INT_MAX = np.int32(2**31 - 1)


def _rotl(x, r):
    return jax.lax.shift_left(x, np.uint32(r)) | jax.lax.shift_right_logical(
        x, np.uint32(32 - r)
    )


def _threefry_xor_bits(lin):
    x0 = jnp.zeros_like(lin)
    x1 = lin + _KS1
    ks = (_KS0, _KS1, _KS2)
    rots = ((13, 15, 26, 6), (17, 29, 16, 24))
    for i in range(5):
        for r in rots[i % 2]:
            x0 = x0 + x1
            x1 = _rotl(x1, r) ^ x0
        x0 = x0 + ks[(i + 1) % 3]
        x1 = x1 + ks[(i + 2) % 3] + np.uint32(i + 1)
    return x0 ^ x1


def _perturbed(p, lin):
    bits = _threefry_xor_bits(lin)
    fb = jax.lax.bitcast_convert_type(
        (bits >> np.uint32(9)) | np.uint32(0x3F800000), jnp.float32
    ) - np.float32(1.0)
    u = jnp.maximum(_TINY, fb * _SCALE + _TINY)
    return -jnp.log(-jnp.log(u)) + jnp.log(jnp.maximum(p, np.float32(1e-30)))


def _main_kernel(p_ref, out_ref, res_ref, m_ref, idx_ref, pat_ref):
    ph = pl.program_id(0)
    s = pl.program_id(1)

    @pl.when((ph == 0) & (s == 0))
    def _init():
        m_ref[...] = jnp.full((1, _B), -jnp.inf, jnp.float32)
        idx_ref[...] = jnp.zeros((1, _B), jnp.int32)
        pat_ref[...] = jnp.zeros((1, _B), jnp.float32)

    lane_u = jax.lax.broadcasted_iota(jnp.uint32, (_CH, _B), 1)
    row_u = jax.lax.broadcasted_iota(jnp.uint32, (_CH, _B), 0)
    row_i = jax.lax.broadcasted_iota(jnp.int32, (_CH, _B), 0)
    base = s * _BV

    @pl.when(ph == 0)
    def _phase0():
        m = m_ref[...]
        idx = idx_ref[...]
        pat = pat_ref[...]
        for k in range(_NCH):
            off = k * _CH
            pch = p_ref[pl.ds(off, _CH), :]
            c_u = base.astype(jnp.uint32) + np.uint32(off) + row_u
            val = _perturbed(pch, lane_u * np.uint32(_V) + c_u)
            cg = base + np.int32(off) + row_i
            bm_c = jnp.max(val, axis=0, keepdims=True)
            is_m = val == bm_c
            idx_c = jnp.min(jnp.where(is_m, cg, _INT_MAX), axis=0, keepdims=True)
            pat_c = jnp.max(
                jnp.where(is_m & (cg == idx_c), pch, np.float32(-1.0)),
                axis=0, keepdims=True,
            )
            upd = bm_c > m
            m = jnp.where(upd, bm_c, m)
            idx = jnp.where(upd, idx_c, idx)
            pat = jnp.where(upd, pat_c, pat)
        m_ref[...] = m
        idx_ref[...] = idx
        pat_ref[...] = pat

    @pl.when(ph == 1)
    def _phase1():
        idx = idx_ref[...]
        pa = pat_ref[...]
        vc = (np.float32(1.0) - pa) / jnp.maximum(pa, np.float32(1e-10))
        for k in range(_NCH):
            off = k * _CH
            pch = p_ref[pl.ds(off, _CH), :]
            cg = base + np.int32(off) + row_i
            vnc = pch / jnp.maximum(np.float32(1.0) - pch, np.float32(1e-10))
            out_ref[pl.ds(off, _CH), :] = jnp.where(cg == idx, vc, vnc)

        @pl.when(s == _NS - 1)
        def _fin():
            res_ref[...] = idx_ref[...]


def kernel(p, alpha, beta):
    del alpha, beta
    pt = p.T
    unc_t, res_t = pl.pallas_call(
        _main_kernel,
        grid=(2, _NS),
        in_specs=[pl.BlockSpec((_BV, _B), lambda ph, s: (s, 0))],
        out_specs=[
            pl.BlockSpec((_BV, _B), lambda ph, s: (jnp.where(ph == 1, s, 0), 0)),
            pl.BlockSpec((1, _B), lambda ph, s: (0, 0)),
        ],
        out_shape=[
            jax.ShapeDtypeStruct((_V, _B), jnp.float32),
            jax.ShapeDtypeStruct((1, _B), jnp.int32),
        ],
        scratch_shapes=[
            pltpu.VMEM((1, _B), jnp.float32),
            pltpu.VMEM((1, _B), jnp.int32),
            pltpu.VMEM((1, _B), jnp.float32),
        ],
    )(pt)
    return (res_t.T, unc_t.T)


# speedup vs baseline: 1.9655x; 1.0914x over previous
import numpy as np
import jax
import jax.numpy as jnp
from jax.experimental import pallas as pl
from jax.experimental.pallas import tpu as pltpu

_B = 128
_V = 100000
_BV = 10000
_NS = _V // _BV
_CH = 200
_NCH = _BV // _CH

_KS0 = np.uint32(0)
_KS1 = np.uint32(42)
_KS2 = np.uint32(np.uint32(0x1BD11BDA) ^ np.uint32(42))
_TINY = np.float32(np.finfo(np.float32).tiny)
_SCALE = np.float32(1.0 - np.finfo(np.float32).tiny)
_INT_MAX = np.int32(2**31 - 1)


def _rotl(x, r):
    return jax.lax.shift_left(x, np.uint32(r)) | jax.lax.shift_right_logical(
        x, np.uint32(32 - r)
    )


def _threefry_xor_bits(lin):
    x0 = jnp.zeros_like(lin)
    x1 = lin + _KS1
    ks = (_KS0, _KS1, _KS2)
    rots = ((13, 15, 26, 6), (17, 29, 16, 24))
    for i in range(5):
        for r in rots[i % 2]:
            x0 = x0 + x1
            x1 = _rotl(x1, r) ^ x0
        x0 = x0 + ks[(i + 1) % 3]
        x1 = x1 + ks[(i + 2) % 3] + np.uint32(i + 1)
    return x0 ^ x1


def _perturbed(p, lin):
    bits = _threefry_xor_bits(lin)
    fb = jax.lax.bitcast_convert_type(
        (bits >> np.uint32(9)) | np.uint32(0x3F800000), jnp.float32
    ) - np.float32(1.0)
    u = jnp.maximum(_TINY, fb * _SCALE + _TINY)
    return -jnp.log(-jnp.log(u)) + jnp.log(jnp.maximum(p, np.float32(1e-30)))


def _main_kernel(p_ref, out_ref, res_ref, m_ref, c_ref, q_ref, if_ref, vc_ref):
    ph = pl.program_id(0)
    s = pl.program_id(1)

    @pl.when((ph == 0) & (s == 0))
    def _init():
        m_ref[...] = jnp.full((_CH, _B), -jnp.inf, jnp.float32)
        c_ref[...] = jnp.zeros((_CH, _B), jnp.int32)
        q_ref[...] = jnp.zeros((_CH, _B), jnp.float32)

    lane_u = jax.lax.broadcasted_iota(jnp.uint32, (_CH, _B), 1)
    row_u = jax.lax.broadcasted_iota(jnp.uint32, (_CH, _B), 0)
    row_i = jax.lax.broadcasted_iota(jnp.int32, (_CH, _B), 0)
    lin0 = lane_u * np.uint32(_V) + row_u
    base_u = (s * _BV).astype(jnp.uint32)
    base_i = s * _BV

    @pl.when(ph == 0)
    def _phase0():
        m = m_ref[...]
        c = c_ref[...]
        q = q_ref[...]
        for k in range(_NCH):
            off = k * _CH
            pch = p_ref[pl.ds(off, _CH), :]
            val = _perturbed(pch, lin0 + (base_u + np.uint32(off)))
            upd = val > m
            m = jnp.where(upd, val, m)
            c = jnp.where(upd, row_i + (base_i + np.int32(off)), c)
            q = jnp.where(upd, pch, q)
        m_ref[...] = m
        c_ref[...] = c
        q_ref[...] = q

    @pl.when((ph == 1) & (s == 0))
    def _fold():
        m = m_ref[...]
        c = c_ref[...]
        q = q_ref[...]
        bm = jnp.max(m, axis=0, keepdims=True)
        is_m = m == bm
        idxf = jnp.min(jnp.where(is_m, c, _INT_MAX), axis=0, keepdims=True)
        qf = jnp.max(
            jnp.where(is_m & (c == idxf), q, np.float32(-1.0)),
            axis=0, keepdims=True,
        )
        if_ref[...] = idxf
        vc_ref[...] = (np.float32(1.0) - qf) / jnp.maximum(qf, np.float32(1e-10))
        res_ref[...] = idxf

    @pl.when(ph == 1)
    def _phase1():
        idxf = if_ref[...]
        vc = vc_ref[...]
        for k in range(_NCH):
            off = k * _CH
            pch = p_ref[pl.ds(off, _CH), :]
            cg = row_i + (base_i + np.int32(off))
            vnc = pch / jnp.maximum(np.float32(1.0) - pch, np.float32(1e-10))
            out_ref[pl.ds(off, _CH), :] = jnp.where(cg == idxf, vc, vnc)


def kernel(p, alpha, beta):
    del alpha, beta
    pt = p.T
    unc_t, res_t = pl.pallas_call(
        _main_kernel,
        grid=(2, _NS),
        in_specs=[pl.BlockSpec((_BV, _B), lambda ph, s: (s, 0))],
        out_specs=[
            pl.BlockSpec((_BV, _B), lambda ph, s: (jnp.where(ph == 1, s, 0), 0)),
            pl.BlockSpec((1, _B), lambda ph, s: (0, 0)),
        ],
        out_shape=[
            jax.ShapeDtypeStruct((_V, _B), jnp.float32),
            jax.ShapeDtypeStruct((1, _B), jnp.int32),
        ],
        scratch_shapes=[
            pltpu.VMEM((_CH, _B), jnp.float32),
            pltpu.VMEM((_CH, _B), jnp.int32),
            pltpu.VMEM((_CH, _B), jnp.float32),
            pltpu.VMEM((1, _B), jnp.int32),
            pltpu.VMEM((1, _B), jnp.float32),
        ],
    )(pt)
    return (res_t.T, unc_t.T)


# speedup vs baseline: 1.9801x; 1.0074x over previous
import numpy as np
import jax
import jax.numpy as jnp
from jax.experimental import pallas as pl
from jax.experimental.pallas import tpu as pltpu

_B = 128
_V = 100000
_BV = 10000
_NS = _V // _BV
_CH = 200
_NCH = _BV // _CH

_KS0 = np.uint32(0)
_KS1 = np.uint32(42)
_KS2 = np.uint32(np.uint32(0x1BD11BDA) ^ np.uint32(42))
_TINY = np.float32(np.finfo(np.float32).tiny)
_SCALE = np.float32(1.0 - np.finfo(np.float32).tiny)
_INT_MAX = np.int32(2**31 - 1)


def _rotl(x, r):
    return jax.lax.shift_left(x, np.uint32(r)) | jax.lax.shift_right_logical(
        x, np.uint32(32 - r)
    )


def _threefry_xor_bits(lin):
    x0 = jnp.zeros_like(lin)
    x1 = lin
    ks = (_KS0, _KS1, _KS2)
    rots = ((13, 15, 26, 6), (17, 29, 16, 24))
    for i in range(5):
        for r in rots[i % 2]:
            x0 = x0 + x1
            x1 = _rotl(x1, r) ^ x0
        x0 = x0 + ks[(i + 1) % 3]
        x1 = x1 + ks[(i + 2) % 3] + np.uint32(i + 1)
    return x0 ^ x1


def _perturbed(p, lin):
    bits = _threefry_xor_bits(lin)
    fb = jax.lax.bitcast_convert_type(
        (bits >> np.uint32(9)) | np.uint32(0x3F800000), jnp.float32
    ) - np.float32(1.0)
    u = jnp.maximum(_TINY, fb * _SCALE + _TINY)
    return -jnp.log(-jnp.log(u)) + jnp.log(jnp.maximum(p, np.float32(1e-30)))


def _main_kernel(p_ref, out_ref, res_ref, m_ref, c_ref, q_ref, if_ref, vc_ref):
    ph = pl.program_id(0)
    s = pl.program_id(1)

    @pl.when((ph == 0) & (s == 0))
    def _init():
        m_ref[...] = jnp.full((_CH, _B), -jnp.inf, jnp.float32)
        c_ref[...] = jnp.zeros((_CH, _B), jnp.int32)
        q_ref[...] = jnp.zeros((_CH, _B), jnp.float32)

    lane_u = jax.lax.broadcasted_iota(jnp.uint32, (_CH, _B), 1)
    row_u = jax.lax.broadcasted_iota(jnp.uint32, (_CH, _B), 0)
    row_i = jax.lax.broadcasted_iota(jnp.int32, (_CH, _B), 0)
    lin0 = lane_u * np.uint32(_V) + (row_u + _KS1)
    base_u = (s * _BV).astype(jnp.uint32)
    base_i = s * _BV

    @pl.when(ph == 0)
    def _phase0():
        m = m_ref[...]
        c = c_ref[...]
        q = q_ref[...]
        for k in range(_NCH):
            off = k * _CH
            pch = p_ref[pl.ds(off, _CH), :]
            val = _perturbed(pch, lin0 + (base_u + np.uint32(off)))
            upd = val > m
            m = jnp.where(upd, val, m)
            c = jnp.where(upd, row_i + (base_i + np.int32(off)), c)
            q = jnp.where(upd, pch, q)
        m_ref[...] = m
        c_ref[...] = c
        q_ref[...] = q

    @pl.when((ph == 1) & (s == 0))
    def _fold():
        m = m_ref[...]
        c = c_ref[...]
        q = q_ref[...]
        bm = jnp.max(m, axis=0, keepdims=True)
        is_m = m == bm
        idxf = jnp.min(jnp.where(is_m, c, _INT_MAX), axis=0, keepdims=True)
        qf = jnp.max(
            jnp.where(is_m & (c == idxf), q, np.float32(-1.0)),
            axis=0, keepdims=True,
        )
        if_ref[...] = idxf
        vc_ref[...] = (np.float32(1.0) - qf) / jnp.maximum(qf, np.float32(1e-10))
        res_ref[...] = idxf

    @pl.when(ph == 1)
    def _phase1():
        idxf = if_ref[...]
        vc = vc_ref[...]
        for k in range(_NCH):
            off = k * _CH
            pch = p_ref[pl.ds(off, _CH), :]
            cg = row_i + (base_i + np.int32(off))
            vnc = pch / jnp.maximum(np.float32(1.0) - pch, np.float32(1e-10))
            out_ref[pl.ds(off, _CH), :] = jnp.where(cg == idxf, vc, vnc)


def kernel(p, alpha, beta):
    del alpha, beta
    pt = p.T
    unc_t, res_t = pl.pallas_call(
        _main_kernel,
        grid=(2, _NS),
        in_specs=[pl.BlockSpec((_BV, _B), lambda ph, s: (s, 0))],
        out_specs=[
            pl.BlockSpec((_BV, _B), lambda ph, s: (jnp.where(ph == 1, s, 0), 0)),
            pl.BlockSpec((1, _B), lambda ph, s: (0, 0)),
        ],
        out_shape=[
            jax.ShapeDtypeStruct((_V, _B), jnp.float32),
            jax.ShapeDtypeStruct((1, _B), jnp.int32),
        ],
        scratch_shapes=[
            pltpu.VMEM((_CH, _B), jnp.float32),
            pltpu.VMEM((_CH, _B), jnp.int32),
            pltpu.VMEM((_CH, _B), jnp.float32),
            pltpu.VMEM((1, _B), jnp.int32),
            pltpu.VMEM((1, _B), jnp.float32),
        ],
    )(pt)
    return (res_t.T, unc_t.T)


# speedup vs baseline: 1.9973x; 1.0087x over previous
import numpy as np
import jax
import jax.numpy as jnp
from jax.experimental import pallas as pl
from jax.experimental.pallas import tpu as pltpu

_B = 128
_V = 100000
_BV = 10000
_NS = _V // _BV
_CH = 200
_NCH = _BV // _CH

_KS0 = np.uint32(0)
_KS1 = np.uint32(42)
_KS2 = np.uint32(np.uint32(0x1BD11BDA) ^ np.uint32(42))
_TINY = np.float32(np.finfo(np.float32).tiny)
_SCALE = np.float32(1.0 - np.finfo(np.float32).tiny)
_INT_MAX = np.int32(2**31 - 1)


def _rotl(x, r):
    return jax.lax.shift_left(x, np.uint32(r)) | jax.lax.shift_right_logical(
        x, np.uint32(32 - r)
    )


def _threefry_xor_bits(lin):
    x0 = jnp.zeros_like(lin)
    x1 = lin
    ks = (_KS0, _KS1, _KS2)
    rots = ((13, 15, 26, 6), (17, 29, 16, 24))
    for i in range(5):
        for r in rots[i % 2]:
            x0 = x0 + x1
            x1 = _rotl(x1, r) ^ x0
        x0 = x0 + ks[(i + 1) % 3]
        x1 = x1 + ks[(i + 2) % 3] + np.uint32(i + 1)
    return x0 ^ x1


def _perturbed(p, lin):
    bits = _threefry_xor_bits(lin)
    fb = jax.lax.bitcast_convert_type(
        (bits >> np.uint32(9)) | np.uint32(0x3F800000), jnp.float32
    ) - np.float32(1.0)
    u = fb + _TINY
    return -jnp.log(-jnp.log(u)) + jnp.log(jnp.maximum(p, np.float32(1e-30)))


def _main_kernel(p_ref, out_ref, res_ref, m_ref, c_ref, q_ref, if_ref, vc_ref):
    ph = pl.program_id(0)
    s = pl.program_id(1)

    @pl.when((ph == 0) & (s == 0))
    def _init():
        m_ref[...] = jnp.full((_CH, _B), -jnp.inf, jnp.float32)
        c_ref[...] = jnp.zeros((_CH, _B), jnp.int32)
        q_ref[...] = jnp.zeros((_CH, _B), jnp.float32)

    lane_u = jax.lax.broadcasted_iota(jnp.uint32, (_CH, _B), 1)
    row_u = jax.lax.broadcasted_iota(jnp.uint32, (_CH, _B), 0)
    row_i = jax.lax.broadcasted_iota(jnp.int32, (_CH, _B), 0)
    lin0 = lane_u * np.uint32(_V) + (row_u + _KS1)
    base_u = (s * _BV).astype(jnp.uint32)
    base_i = s * _BV

    @pl.when(ph == 0)
    def _phase0():
        m = m_ref[...]
        c = c_ref[...]
        q = q_ref[...]
        for k in range(_NCH):
            off = k * _CH
            pch = p_ref[pl.ds(off, _CH), :]
            val = _perturbed(pch, lin0 + (base_u + np.uint32(off)))
            upd = val > m
            m = jnp.where(upd, val, m)
            c = jnp.where(upd, jnp.zeros((_CH, _B), jnp.int32) + (base_i + np.int32(off)), c)
            q = jnp.where(upd, pch, q)
        m_ref[...] = m
        c_ref[...] = c
        q_ref[...] = q

    @pl.when((ph == 1) & (s == 0))
    def _fold():
        m = m_ref[...]
        c = c_ref[...] + row_i
        q = q_ref[...]
        bm = jnp.max(m, axis=0, keepdims=True)
        is_m = m == bm
        idxf = jnp.min(jnp.where(is_m, c, _INT_MAX), axis=0, keepdims=True)
        qf = jnp.max(
            jnp.where(is_m & (c == idxf), q, np.float32(-1.0)),
            axis=0, keepdims=True,
        )
        if_ref[...] = idxf
        vc_ref[...] = (np.float32(1.0) - qf) / jnp.maximum(qf, np.float32(1e-10))
        res_ref[...] = idxf

    @pl.when(ph == 1)
    def _phase1():
        idxf = if_ref[...]
        vc = vc_ref[...]
        for k in range(_NCH):
            off = k * _CH
            pch = p_ref[pl.ds(off, _CH), :]
            cg = row_i + (base_i + np.int32(off))
            vnc = pch / jnp.maximum(np.float32(1.0) - pch, np.float32(1e-10))
            out_ref[pl.ds(off, _CH), :] = jnp.where(cg == idxf, vc, vnc)


def kernel(p, alpha, beta):
    del alpha, beta
    pt = p.T
    unc_t, res_t = pl.pallas_call(
        _main_kernel,
        grid=(2, _NS),
        in_specs=[pl.BlockSpec((_BV, _B), lambda ph, s: (s, 0))],
        out_specs=[
            pl.BlockSpec((_BV, _B), lambda ph, s: (jnp.where(ph == 1, s, 0), 0)),
            pl.BlockSpec((1, _B), lambda ph, s: (0, 0)),
        ],
        out_shape=[
            jax.ShapeDtypeStruct((_V, _B), jnp.float32),
            jax.ShapeDtypeStruct((1, _B), jnp.int32),
        ],
        scratch_shapes=[
            pltpu.VMEM((_CH, _B), jnp.float32),
            pltpu.VMEM((_CH, _B), jnp.int32),
            pltpu.VMEM((_CH, _B), jnp.float32),
            pltpu.VMEM((1, _B), jnp.int32),
            pltpu.VMEM((1, _B), jnp.float32),
        ],
    )(pt)
    return (res_t.T, unc_t.T)
